# baseline (speedup 1.0000x reference)
import functools

import jax
import jax.numpy as jnp
from jax import lax
from jax.experimental import pallas as pl
from jax.experimental.pallas import tpu as pltpu
from jax.experimental.pallas import tpu_sc as plsc

P_DIM = 128
K_DIM = 32
N_ROWS = 1000000
B_DIM = 16384
_NC, _NS = 2, 16
_NW = _NC * _NS
_CHUNK = 1024
_LASTW = 512
_COLS_MAIN = 999936
_ITEMS = 977
_OUT_ROWS = B_DIM + 128
_NSLOT = 4


def _sc_gather_body(idx_hbm, zt_hbm, out_hbm, idx_v, hits_v, buf_v, clc_v,
                    stage_v, sin0, sin1, so0, so1, so2, so3):
    t = lax.axis_index("s") * _NC + lax.axis_index("c")
    start_item = 30 * t + jnp.minimum(t, 17)
    n_items = jnp.where(t < 17, 31, 30)
    lo = start_item * _CHUNK
    hi = jnp.minimum((start_item + n_items) * _CHUNK, _COLS_MAIN)
    out_sems = (so0, so1, so2, so3)

    iota16 = lax.iota(jnp.int32, 16)

    def fire_load(j):
        item = start_item + j
        c0 = item * _CHUNK
        is_last = item == _ITEMS - 1
        for p, sem in ((0, sin0), (1, sin1)):
            @pl.when((j % 2 == p) & jnp.logical_not(is_last))
            def _(p=p, sem=sem, c0=c0):
                pltpu.async_copy(
                    zt_hbm.at[:, pl.ds(c0, _CHUNK)],
                    buf_v.at[pl.ds(p * K_DIM, K_DIM)], sem)

            @pl.when((j % 2 == p) & is_last)
            def _(p=p, sem=sem, c0=c0):
                pltpu.async_copy(
                    zt_hbm.at[:, pl.ds(c0, _LASTW)],
                    buf_v.at[pl.ds(p * K_DIM, K_DIM), pl.ds(0, _LASTW)], sem)

    def wait_load(j):
        item = start_item + j
        c0 = item * _CHUNK
        is_last = item == _ITEMS - 1
        for p, sem in ((0, sin0), (1, sin1)):
            @pl.when((j % 2 == p) & jnp.logical_not(is_last))
            def _(p=p, sem=sem, c0=c0):
                pltpu.make_async_copy(
                    zt_hbm.at[:, pl.ds(c0, _CHUNK)],
                    buf_v.at[pl.ds(p * K_DIM, K_DIM)], sem).wait()

            @pl.when((j % 2 == p) & is_last)
            def _(p=p, sem=sem, c0=c0):
                pltpu.make_async_copy(
                    zt_hbm.at[:, pl.ds(c0, _LASTW)],
                    buf_v.at[pl.ds(p * K_DIM, K_DIM), pl.ds(0, _LASTW)],
                    sem).wait()

    fire_load(0)

    pltpu.sync_copy(idx_hbm, idx_v)

    def scan_body(v, cnt):
        iv = idx_v[pl.ds(v * 16, 16)]
        m = (iv >= lo) & (iv < hi)
        packed = ((v * 16 + iota16) << 15) | (iv - lo)
        _, sv = plsc.sort_key_val(jnp.where(m, iota16, 9999), packed)
        plsc.store_scatter(hits_v, [cnt + iota16], sv)
        return cnt + plsc.all_reduce_population_count(m)[0]

    cnt = lax.fori_loop(0, B_DIM // 16, scan_body, jnp.int32(0))
    plsc.store_scatter(hits_v, [cnt + iota16], iota16 * 0 + 0x7FFF)

    def chunk_step(j, g):
        item = start_item + j

        @pl.when(j + 1 < n_items)
        def _():
            fire_load(j + 1)

        wait_load(j)

        c0 = item * _CHUNK
        is_last = item == _ITEMS - 1
        width = jnp.where(is_last, _LASTW, _CHUNK)
        c0l = c0 - lo
        pbase = (j % 2) * K_DIM

        def resc_body(h, ccnt):
            pk = hits_v[pl.ds(h * 16, 16)]
            il = pk & 0x7FFF
            m2 = (il >= c0l) & (il < c0l + width)
            pk2 = ((pk >> 15) << 10) | (il - c0l)
            _, sv = plsc.sort_key_val(jnp.where(m2, iota16, 9999), pk2)
            plsc.store_scatter(clc_v, [ccnt + iota16], sv)
            return ccnt + plsc.all_reduce_population_count(m2)[0]

        ccnt = lax.fori_loop(0, (cnt + 15) // 16, resc_body, jnp.int32(0))
        plsc.store_scatter(clc_v, [ccnt + iota16],
                           iota16 * 0 + ((B_DIM + t) << 10))

        def ext_body(e, gg):
            pk2 = clc_v[pl.ds(e * 16, 16)]
            lc = pk2 & 0x3FF
            bv = pk2 >> 10
            slot = gg % _NSLOT
            for s, sem in enumerate(out_sems):
                @pl.when((slot == s) & (gg >= _NSLOT))
                def _(s=s, sem=sem):
                    pltpu.make_async_copy(
                        stage_v.at[pl.ds(s * 16, 16)],
                        out_hbm.at[bv], sem).wait()
            for k in range(K_DIM):
                rk = iota16 * 0 + (pbase + k)
                vk = plsc.load_gather(buf_v, [rk, lc])
                plsc.store_scatter(stage_v, [slot * 16 + iota16, rk * 0 + k],
                                   vk)
            for s, sem in enumerate(out_sems):
                @pl.when(slot == s)
                def _(s=s, sem=sem):
                    pltpu.async_copy(stage_v.at[pl.ds(s * 16, 16)],
                                     out_hbm.at[bv], sem)
            return gg + 1

        return lax.fori_loop(0, (ccnt + 15) // 16, ext_body, g)

    g = lax.fori_loop(0, n_items, chunk_step, jnp.int32(0))

    dump_bv = iota16 * 0 + (B_DIM + t)
    for s, sem in enumerate(out_sems):
        @pl.when(g > s)
        def _(s=s, sem=sem):
            pltpu.make_async_copy(stage_v.at[pl.ds(s * 16, 16)],
                                  out_hbm.at[dump_bv], sem).wait()


@functools.partial(
    pl.kernel,
    out_type=jax.ShapeDtypeStruct((_OUT_ROWS, K_DIM), jnp.float32),
    mesh=plsc.VectorSubcoreMesh(core_axis_name="c", subcore_axis_name="s"),
    scratch_types=[
        pltpu.VMEM((B_DIM,), jnp.int32),
        pltpu.VMEM((B_DIM + 128,), jnp.int32),
        pltpu.VMEM((2 * K_DIM, _CHUNK), jnp.float32),
        pltpu.VMEM((B_DIM + 128,), jnp.int32),
        pltpu.VMEM((_NSLOT * 16, K_DIM), jnp.float32),
        pltpu.SemaphoreType.DMA,
        pltpu.SemaphoreType.DMA,
        pltpu.SemaphoreType.DMA,
        pltpu.SemaphoreType.DMA,
        pltpu.SemaphoreType.DMA,
        pltpu.SemaphoreType.DMA,
    ],
    compiler_params=pltpu.CompilerParams(
        needs_layout_passes=False,
        use_tc_tiling_on_sc=False,
    ),
)
def _sc_gather(idx_hbm, zt_hbm, out_hbm, idx_v, hits_v, buf_v, clc_v,
               stage_v, sin0, sin1, so0, so1, so2, so3):
    _sc_gather_body(idx_hbm, zt_hbm, out_hbm, idx_v, hits_v, buf_v, clc_v,
                    stage_v, sin0, sin1, so0, so1, so2, so3)


_BLK_B = 2048


def _mm_body(zp_ref, idx_ref, ztail_ref, w_ref, mu_ref, o_ref):
    zi = zp_ref[...]
    iv = idx_ref[...]
    fix = iv >= _COLS_MAIN
    lc = iv - _COLS_MAIN
    oh = (
        lc == lax.broadcasted_iota(jnp.int32, (_BLK_B, N_ROWS - _COLS_MAIN), 1)
    ) & fix
    zi_fix = lax.dot_general(
        oh.astype(jnp.float32),
        ztail_ref[...],
        dimension_numbers=(((1,), (0,)), ((), ())),
        preferred_element_type=jnp.float32,
    )
    zi_use = jnp.where(fix, zi_fix, zi)
    o_ref[...] = (
        lax.dot_general(
            zi_use,
            w_ref[...],
            dimension_numbers=(((1,), (1,)), ((), ())),
            preferred_element_type=jnp.float32,
        )
        + mu_ref[...]
    )


def _matmul(zi_pad, idx_col, ztail, W, mu2d):
    ntail = N_ROWS - _COLS_MAIN
    return pl.pallas_call(
        _mm_body,
        grid=(B_DIM // _BLK_B,),
        in_specs=[
            pl.BlockSpec((_BLK_B, K_DIM), lambda i: (i, 0)),
            pl.BlockSpec((_BLK_B, 1), lambda i: (i, 0)),
            pl.BlockSpec((ntail, K_DIM), lambda i: (0, 0)),
            pl.BlockSpec((P_DIM, K_DIM), lambda i: (0, 0)),
            pl.BlockSpec((1, P_DIM), lambda i: (0, 0)),
        ],
        out_specs=pl.BlockSpec((_BLK_B, P_DIM), lambda i: (i, 0)),
        out_shape=jax.ShapeDtypeStruct((B_DIM, P_DIM), jnp.float32),
    )(zi_pad, idx_col, ztail, W, mu2d)


def kernel(X, indices, z, W, mu):
    zt = z.T
    zi_pad = _sc_gather(indices, zt)
    idx_col = indices.reshape(B_DIM, 1)
    ztail = z[_COLS_MAIN:, :]
    return _matmul(zi_pad, idx_col, ztail, W, mu.reshape(1, P_DIM))


# speedup vs baseline: 21.8038x; 21.8038x over previous
import functools

import jax
import jax.numpy as jnp
from jax import lax
from jax.experimental import pallas as pl
from jax.experimental.pallas import tpu as pltpu
from jax.experimental.pallas import tpu_sc as plsc

P_DIM = 128
K_DIM = 32
N_ROWS = 1000000
B_DIM = 16384
_NC, _NS = 2, 16
_NW = _NC * _NS
_CHUNK = 1024
_LASTW = 512
_COLS_MAIN = 999936
_ITEMS = 977
_OUT_ROWS = B_DIM + 128
_NSLOT = 4


def _sc_gather_body(idx_hbm, zt_hbm, out_hbm, idx_v, hits_v, buf_v, clc_v,
                    stage_v, sin0, sin1, so0, so1, so2, so3):
    t = lax.axis_index("s") * _NC + lax.axis_index("c")
    start_item = 30 * t + jnp.minimum(t, 17)
    n_items = jnp.where(t < 17, 31, 30)
    lo = start_item * _CHUNK
    hi = jnp.minimum((start_item + n_items) * _CHUNK, _COLS_MAIN)
    out_sems = (so0, so1, so2, so3)

    iota16 = lax.iota(jnp.int32, 16)

    def fire_load(j):
        item = start_item + j
        c0 = item * _CHUNK
        is_last = item == _ITEMS - 1
        for p, sem in ((0, sin0), (1, sin1)):
            @pl.when((j % 2 == p) & jnp.logical_not(is_last))
            def _(p=p, sem=sem, c0=c0):
                pltpu.async_copy(
                    zt_hbm.at[:, pl.ds(c0, _CHUNK)],
                    buf_v.at[pl.ds(p * K_DIM, K_DIM)], sem)

            @pl.when((j % 2 == p) & is_last)
            def _(p=p, sem=sem, c0=c0):
                pltpu.async_copy(
                    zt_hbm.at[:, pl.ds(c0, _LASTW)],
                    buf_v.at[pl.ds(p * K_DIM, K_DIM), pl.ds(0, _LASTW)], sem)

    def wait_load(j):
        item = start_item + j
        c0 = item * _CHUNK
        is_last = item == _ITEMS - 1
        for p, sem in ((0, sin0), (1, sin1)):
            @pl.when((j % 2 == p) & jnp.logical_not(is_last))
            def _(p=p, sem=sem, c0=c0):
                pltpu.make_async_copy(
                    zt_hbm.at[:, pl.ds(c0, _CHUNK)],
                    buf_v.at[pl.ds(p * K_DIM, K_DIM)], sem).wait()

            @pl.when((j % 2 == p) & is_last)
            def _(p=p, sem=sem, c0=c0):
                pltpu.make_async_copy(
                    zt_hbm.at[:, pl.ds(c0, _LASTW)],
                    buf_v.at[pl.ds(p * K_DIM, K_DIM), pl.ds(0, _LASTW)],
                    sem).wait()

    fire_load(0)

    pltpu.sync_copy(idx_hbm, idx_v)

    def scan_body(v, cnt):
        iv = idx_v[pl.ds(v * 16, 16)]
        m = (iv >= lo) & (iv < hi)
        packed = ((v * 16 + iota16) << 15) | (iv - lo)
        _, sv = plsc.sort_key_val(jnp.where(m, iota16, 9999), packed)
        plsc.store_scatter(hits_v, [cnt + iota16], sv)
        return cnt + plsc.all_reduce_population_count(m)[0]

    cnt = lax.fori_loop(0, B_DIM // 16, scan_body, jnp.int32(0))
    plsc.store_scatter(hits_v, [cnt + iota16], iota16 * 0 + 0x7FFF)

    def chunk_step(j, g):
        item = start_item + j

        @pl.when(j + 1 < n_items)
        def _():
            fire_load(j + 1)

        wait_load(j)

        c0 = item * _CHUNK
        is_last = item == _ITEMS - 1
        width = jnp.where(is_last, _LASTW, _CHUNK)
        c0l = c0 - lo
        pbase = (j % 2) * K_DIM

        def resc_body(h, ccnt):
            pk = hits_v[pl.ds(h * 16, 16)]
            il = pk & 0x7FFF
            m2 = (il >= c0l) & (il < c0l + width)
            pk2 = ((pk >> 15) << 10) | (il - c0l)
            _, sv = plsc.sort_key_val(jnp.where(m2, iota16, 9999), pk2)
            plsc.store_scatter(clc_v, [ccnt + iota16], sv)
            return ccnt + plsc.all_reduce_population_count(m2)[0]

        ccnt = lax.fori_loop(0, (cnt + 15) // 16, resc_body, jnp.int32(0))
        plsc.store_scatter(clc_v, [ccnt + iota16],
                           iota16 * 0 + ((B_DIM + t) << 10))

        def ext_body(e, gg):
            pk2 = clc_v[pl.ds(e * 16, 16)]
            lc = pk2 & 0x3FF
            bv = pk2 >> 10
            slot = gg % _NSLOT
            for s, sem in enumerate(out_sems):
                @pl.when((slot == s) & (gg >= _NSLOT))
                def _(s=s, sem=sem):
                    pltpu.make_async_copy(
                        stage_v.at[pl.ds(s * 16, 16)],
                        out_hbm.at[bv], sem).wait()
            for k in range(K_DIM):
                rk = iota16 * 0 + (pbase + k)
                vk = plsc.load_gather(buf_v, [rk, lc])
                plsc.store_scatter(stage_v, [slot * 16 + iota16, rk * 0 + k],
                                   vk)
            for s, sem in enumerate(out_sems):
                @pl.when(slot == s)
                def _(s=s, sem=sem):
                    pltpu.async_copy(stage_v.at[pl.ds(s * 16, 16)],
                                     out_hbm.at[bv], sem)
            return gg + 1

        return lax.fori_loop(0, (ccnt + 15) // 16, ext_body, g)

    g = lax.fori_loop(0, n_items, chunk_step, jnp.int32(0))

    dump_bv = iota16 * 0 + (B_DIM + t)
    for s, sem in enumerate(out_sems):
        @pl.when(g > s)
        def _(s=s, sem=sem):
            pltpu.make_async_copy(stage_v.at[pl.ds(s * 16, 16)],
                                  out_hbm.at[dump_bv], sem).wait()


@functools.partial(
    pl.kernel,
    out_type=jax.ShapeDtypeStruct((_OUT_ROWS, P_DIM), jnp.float32),
    mesh=plsc.VectorSubcoreMesh(core_axis_name="c", subcore_axis_name="s"),
    scratch_types=[
        pltpu.VMEM((B_DIM,), jnp.int32),
        pltpu.VMEM((B_DIM + 128,), jnp.int32),
        pltpu.VMEM((2 * K_DIM, _CHUNK), jnp.float32),
        pltpu.VMEM((B_DIM + 128,), jnp.int32),
        pltpu.VMEM((_NSLOT * 16, P_DIM), jnp.float32),
        pltpu.SemaphoreType.DMA,
        pltpu.SemaphoreType.DMA,
        pltpu.SemaphoreType.DMA,
        pltpu.SemaphoreType.DMA,
        pltpu.SemaphoreType.DMA,
        pltpu.SemaphoreType.DMA,
    ],
    compiler_params=pltpu.CompilerParams(needs_layout_passes=False),
)
def _sc_gather(idx_hbm, zt_hbm, out_hbm, idx_v, hits_v, buf_v, clc_v,
               stage_v, sin0, sin1, so0, so1, so2, so3):
    _sc_gather_body(idx_hbm, zt_hbm, out_hbm, idx_v, hits_v, buf_v, clc_v,
                    stage_v, sin0, sin1, so0, so1, so2, so3)


_BLK_B = 4096


def _mm_body(zp_ref, idx_ref, ztail_ref, w_ref, mu_ref, o_ref):
    zi = zp_ref[:, :K_DIM]
    iv = idx_ref[...]
    fix = iv >= _COLS_MAIN
    lc = iv - _COLS_MAIN
    oh = (
        lc == lax.broadcasted_iota(jnp.int32, (_BLK_B, N_ROWS - _COLS_MAIN), 1)
    ) & fix
    zi_fix = lax.dot_general(
        oh.astype(jnp.float32),
        ztail_ref[...],
        dimension_numbers=(((1,), (0,)), ((), ())),
        preferred_element_type=jnp.float32,
    )
    zi_use = jnp.where(fix, zi_fix, zi)
    o_ref[...] = (
        lax.dot_general(
            zi_use,
            w_ref[...],
            dimension_numbers=(((1,), (1,)), ((), ())),
            preferred_element_type=jnp.float32,
        )
        + mu_ref[...]
    )


def _matmul(zi_pad, idx_col, ztail, W, mu2d):
    ntail = N_ROWS - _COLS_MAIN
    return pl.pallas_call(
        _mm_body,
        grid=(B_DIM // _BLK_B,),
        in_specs=[
            pl.BlockSpec((_BLK_B, P_DIM), lambda i: (i, 0)),
            pl.BlockSpec((_BLK_B, 1), lambda i: (i, 0)),
            pl.BlockSpec((ntail, K_DIM), lambda i: (0, 0)),
            pl.BlockSpec((P_DIM, K_DIM), lambda i: (0, 0)),
            pl.BlockSpec((1, P_DIM), lambda i: (0, 0)),
        ],
        out_specs=pl.BlockSpec((_BLK_B, P_DIM), lambda i: (i, 0)),
        out_shape=jax.ShapeDtypeStruct((B_DIM, P_DIM), jnp.float32),
    )(zi_pad, idx_col, ztail, W, mu2d)


def kernel(X, indices, z, W, mu):
    zt = z.T
    zi_pad = _sc_gather(indices, zt)
    idx_col = indices.reshape(B_DIM, 1)
    ztail = z[_COLS_MAIN:, :]
    return _matmul(zi_pad, idx_col, ztail, W, mu.reshape(1, P_DIM))
